# baseline (speedup 1.0000x reference)
import functools

import jax
import jax.numpy as jnp
from jax import lax
from jax.experimental import pallas as pl
from jax.experimental.pallas import tpu as pltpu
from jax.experimental.pallas import tpu_sc as plsc

_NT = 20
_NCORNER = 8
_NLUT = 480
_D = 256
_IDX_R = 2048
_CH = 128


def _lut_body(tab_ref, w_ref, b_ref, lut_ref):
    lut48 = jax.lax.dot_general(
        tab_ref[:], w_ref[:], (((1,), (1,)), ((), ())),
        preferred_element_type=jnp.float32)
    k = jax.lax.broadcasted_iota(jnp.int32, (_NLUT, 1), 0)
    col = jax.lax.broadcasted_iota(jnp.int32, (_NLUT, 48), 1)
    corner = k < 192
    ke = k - 192
    sel_s = jnp.where(corner, k // 24, 19 + ke // 24)
    sel_p = jnp.where(corner, 8 + (k % 24) // 3, 31 + (ke % 24) // 2)
    sel_o = jnp.where(corner, 16 + k % 3, 43 + ke % 2)
    hot3 = ((col == sel_s) | (col == sel_p) | (col == sel_o)).astype(
        jnp.float32)
    lut_ref[:] = jax.lax.dot_general(
        hot3, lut48, (((1,), (0,)), ((), ())),
        preferred_element_type=jnp.float32) + b_ref[:]


def _idx_body(slot_ref, piece_ref, or_ref, idx_ref):
    tok = jax.lax.broadcasted_iota(jnp.int32, idx_ref.shape, 1)
    s, p, o = slot_ref[:], piece_ref[:], or_ref[:]
    idx_ref[:] = jnp.where(tok < _NCORNER,
                           s * 24 + p * 3 + o,
                           192 + s * 24 + p * 2 + o)


_NC = 2
_NS = 16


def _make_sc_gather(rows):
    nw = _NC * _NS
    per_w = rows // nw
    n_ch = per_w // _CH
    mesh = plsc.VectorSubcoreMesh(
        core_axis_name="c", subcore_axis_name="s",
        num_cores=_NC, num_subcores=_NS)

    @functools.partial(
        pl.kernel, mesh=mesh,
        out_type=jax.ShapeDtypeStruct((rows, _D), jnp.float32),
        scratch_types=[
            pltpu.VMEM((per_w,), jnp.int32),
            pltpu.VMEM((_CH, _D), jnp.float32),
            pltpu.SemaphoreType.DMA,
        ],
    )
    def sc_gather(lut_hbm, idx_hbm, out_hbm, idx_v, buf, sem):
        wid = lax.axis_index("s") * _NC + lax.axis_index("c")
        base = wid * per_w
        pltpu.sync_copy(idx_hbm.at[pl.ds(base, per_w)], idx_v)

        def chunk(c, carry):
            off = pl.multiple_of(c * _CH, _CH)
            pltpu.async_copy(
                lut_hbm.at[idx_v.at[pl.ds(off, _CH)]], buf, sem).wait()
            pltpu.sync_copy(buf, out_hbm.at[pl.ds(base + off, _CH)])
            return carry

        lax.fori_loop(0, n_ch, chunk, 0)

    return sc_gather


def kernel(slot_ids, piece_ids, orientations, corner_slot_emb,
           corner_piece_emb, corner_orient_emb, edge_slot_emb, edge_piece_emb,
           edge_orient_emb, proj_W, proj_b):
    bsz = slot_ids.shape[0]
    tab = jnp.zeros((48, 128), jnp.float32)
    tab = tab.at[0:8, 0:42].set(corner_slot_emb)
    tab = tab.at[8:16, 42:84].set(corner_piece_emb)
    tab = tab.at[16:19, 84:128].set(corner_orient_emb)
    tab = tab.at[19:31, 0:42].set(edge_slot_emb)
    tab = tab.at[31:43, 42:84].set(edge_piece_emb)
    tab = tab.at[43:45, 84:128].set(edge_orient_emb)
    bias = proj_b.reshape(1, _D)

    lut = pl.pallas_call(
        _lut_body,
        out_shape=jax.ShapeDtypeStruct((_NLUT, _D), jnp.float32),
    )(tab, proj_W, bias)

    idx = pl.pallas_call(
        _idx_body,
        grid=(bsz // _IDX_R,),
        in_specs=[pl.BlockSpec((_IDX_R, _NT), lambda i: (i, 0))] * 3,
        out_specs=pl.BlockSpec((_IDX_R, _NT), lambda i: (i, 0)),
        out_shape=jax.ShapeDtypeStruct((bsz, _NT), jnp.int32),
    )(slot_ids, piece_ids, orientations)

    rows = bsz * _NT
    out = _make_sc_gather(rows)(lut, idx.reshape(rows))
    return out.reshape(bsz, _NT, _D)


# speedup vs baseline: 1.0018x; 1.0018x over previous
import functools

import jax
import jax.numpy as jnp
from jax import lax
from jax.experimental import pallas as pl
from jax.experimental.pallas import tpu as pltpu
from jax.experimental.pallas import tpu_sc as plsc

_NT = 20
_NCORNER = 8
_NLUT = 480
_D = 256
_IDX_R = 2048
_CH = 128


def _lut_body(tab_ref, w_ref, b_ref, lut_ref):
    lut48 = jax.lax.dot_general(
        tab_ref[:], w_ref[:], (((1,), (1,)), ((), ())),
        preferred_element_type=jnp.float32)
    k = jax.lax.broadcasted_iota(jnp.int32, (_NLUT, 1), 0)
    col = jax.lax.broadcasted_iota(jnp.int32, (_NLUT, 48), 1)
    corner = k < 192
    ke = k - 192
    sel_s = jnp.where(corner, k // 24, 19 + ke // 24)
    sel_p = jnp.where(corner, 8 + (k % 24) // 3, 31 + (ke % 24) // 2)
    sel_o = jnp.where(corner, 16 + k % 3, 43 + ke % 2)
    hot3 = ((col == sel_s) | (col == sel_p) | (col == sel_o)).astype(
        jnp.float32)
    lut_ref[:] = jax.lax.dot_general(
        hot3, lut48, (((1,), (0,)), ((), ())),
        preferred_element_type=jnp.float32) + b_ref[:]


def _idx_body(slot_ref, piece_ref, or_ref, idx_ref):
    tok = jax.lax.broadcasted_iota(jnp.int32, idx_ref.shape, 1)
    s, p, o = slot_ref[:], piece_ref[:], or_ref[:]
    idx_ref[:] = jnp.where(tok < _NCORNER,
                           s * 24 + p * 3 + o,
                           192 + s * 24 + p * 2 + o)


_NC = 2
_NS = 16


def _make_sc_gather(rows):
    nw = _NC * _NS
    per_w = rows // nw
    n_ch = per_w // _CH
    mesh = plsc.VectorSubcoreMesh(
        core_axis_name="c", subcore_axis_name="s",
        num_cores=_NC, num_subcores=_NS)

    n_groups = n_ch // 2

    @functools.partial(
        pl.kernel, mesh=mesh,
        out_type=jax.ShapeDtypeStruct((rows, _D), jnp.float32),
        scratch_types=[
            pltpu.VMEM((per_w,), jnp.int32),
            pltpu.VMEM((_CH, _D), jnp.float32),
            pltpu.VMEM((_CH, _D), jnp.float32),
            pltpu.SemaphoreType.DMA,
            pltpu.SemaphoreType.DMA,
            pltpu.SemaphoreType.DMA,
            pltpu.SemaphoreType.DMA,
        ],
    )
    def sc_gather(lut_hbm, idx_hbm, out_hbm, idx_v, buf0, buf1,
                  gsem0, gsem1, ssem0, ssem1):
        wid = lax.axis_index("s") * _NC + lax.axis_index("c")
        base = wid * per_w
        pltpu.sync_copy(idx_hbm.at[pl.ds(base, per_w)], idx_v)

        bufs = (buf0, buf1)
        gsems = (gsem0, gsem1)
        ssems = (ssem0, ssem1)

        def gather_src(c):
            off = pl.multiple_of(c * _CH, _CH)
            return lut_hbm.at[idx_v.at[pl.ds(off, _CH)]]

        def out_dst(c):
            off = pl.multiple_of(c * _CH, _CH)
            return out_hbm.at[pl.ds(base + off, _CH)]

        for b in range(2):
            pltpu.async_copy(gather_src(b), bufs[b], gsems[b])

        def group(g, carry):
            for b in range(2):
                c = g * 2 + b
                pltpu.make_async_copy(gather_src(c), bufs[b], gsems[b]).wait()
                pltpu.async_copy(bufs[b], out_dst(c), ssems[b])
            for b in range(2):
                c = g * 2 + b

                @pl.when(g < n_groups - 1)
                def _():
                    pltpu.make_async_copy(bufs[b], out_dst(c),
                                          ssems[b]).wait()
                    pltpu.async_copy(gather_src(c + 2), bufs[b], gsems[b])

            return carry

        lax.fori_loop(0, n_groups, group, 0)
        for b in range(2):
            c = (n_groups - 1) * 2 + b
            pltpu.make_async_copy(bufs[b], out_dst(c), ssems[b]).wait()

    return sc_gather


def kernel(slot_ids, piece_ids, orientations, corner_slot_emb,
           corner_piece_emb, corner_orient_emb, edge_slot_emb, edge_piece_emb,
           edge_orient_emb, proj_W, proj_b):
    bsz = slot_ids.shape[0]
    tab = jnp.zeros((48, 128), jnp.float32)
    tab = tab.at[0:8, 0:42].set(corner_slot_emb)
    tab = tab.at[8:16, 42:84].set(corner_piece_emb)
    tab = tab.at[16:19, 84:128].set(corner_orient_emb)
    tab = tab.at[19:31, 0:42].set(edge_slot_emb)
    tab = tab.at[31:43, 42:84].set(edge_piece_emb)
    tab = tab.at[43:45, 84:128].set(edge_orient_emb)
    bias = proj_b.reshape(1, _D)

    lut = pl.pallas_call(
        _lut_body,
        out_shape=jax.ShapeDtypeStruct((_NLUT, _D), jnp.float32),
    )(tab, proj_W, bias)

    idx = pl.pallas_call(
        _idx_body,
        grid=(bsz // _IDX_R,),
        in_specs=[pl.BlockSpec((_IDX_R, _NT), lambda i: (i, 0))] * 3,
        out_specs=pl.BlockSpec((_IDX_R, _NT), lambda i: (i, 0)),
        out_shape=jax.ShapeDtypeStruct((bsz, _NT), jnp.int32),
    )(slot_ids, piece_ids, orientations)

    rows = bsz * _NT
    out = _make_sc_gather(rows)(lut, idx.reshape(rows))
    return out.reshape(bsz, _NT, _D)


# speedup vs baseline: 1.0103x; 1.0084x over previous
import functools

import jax
import jax.numpy as jnp
from jax import lax
from jax.experimental import pallas as pl
from jax.experimental.pallas import tpu as pltpu
from jax.experimental.pallas import tpu_sc as plsc

_NT = 20
_NCORNER = 8
_NLUT = 480
_D = 256
_IDX_R = 2048
_CH = 128


def _lut_body(tab_ref, w_ref, b_ref, lut_ref):
    lut48 = jax.lax.dot_general(
        tab_ref[:], w_ref[:], (((1,), (1,)), ((), ())),
        preferred_element_type=jnp.float32)
    k = jax.lax.broadcasted_iota(jnp.int32, (_NLUT, 1), 0)
    col = jax.lax.broadcasted_iota(jnp.int32, (_NLUT, 48), 1)
    corner = k < 192
    ke = k - 192
    sel_s = jnp.where(corner, k // 24, 19 + ke // 24)
    sel_p = jnp.where(corner, 8 + (k % 24) // 3, 31 + (ke % 24) // 2)
    sel_o = jnp.where(corner, 16 + k % 3, 43 + ke % 2)
    hot3 = ((col == sel_s) | (col == sel_p) | (col == sel_o)).astype(
        jnp.float32)
    lut_ref[:] = jax.lax.dot_general(
        hot3, lut48, (((1,), (0,)), ((), ())),
        preferred_element_type=jnp.float32) + b_ref[:]


_NC = 2
_NS = 16


def _make_sc_gather(rows):
    nw = _NC * _NS
    per_w = rows // nw
    n_ch = per_w // _CH
    mesh = plsc.VectorSubcoreMesh(
        core_axis_name="c", subcore_axis_name="s",
        num_cores=_NC, num_subcores=_NS)

    n_groups = n_ch // 2

    @functools.partial(
        pl.kernel, mesh=mesh,
        out_type=jax.ShapeDtypeStruct((rows, _D), jnp.float32),
        scratch_types=[
            pltpu.VMEM((per_w,), jnp.int32),
            pltpu.VMEM((_CH, _D), jnp.float32),
            pltpu.VMEM((_CH, _D), jnp.float32),
            pltpu.SemaphoreType.DMA,
            pltpu.SemaphoreType.DMA,
            pltpu.SemaphoreType.DMA,
            pltpu.SemaphoreType.DMA,
        ],
    )
    def sc_gather(lut_hbm, idx_hbm, out_hbm, idx_v, buf0, buf1,
                  gsem0, gsem1, ssem0, ssem1):
        wid = lax.axis_index("s") * _NC + lax.axis_index("c")
        base = wid * per_w
        pltpu.sync_copy(idx_hbm.at[pl.ds(base, per_w)], idx_v)

        bufs = (buf0, buf1)
        gsems = (gsem0, gsem1)
        ssems = (ssem0, ssem1)

        def gather_src(c):
            off = pl.multiple_of(c * _CH, _CH)
            return lut_hbm.at[idx_v.at[pl.ds(off, _CH)]]

        def out_dst(c):
            off = pl.multiple_of(c * _CH, _CH)
            return out_hbm.at[pl.ds(base + off, _CH)]

        for b in range(2):
            pltpu.async_copy(gather_src(b), bufs[b], gsems[b])

        def group(g, carry):
            for b in range(2):
                c = g * 2 + b
                pltpu.make_async_copy(gather_src(c), bufs[b], gsems[b]).wait()
                pltpu.async_copy(bufs[b], out_dst(c), ssems[b])
            for b in range(2):
                c = g * 2 + b

                @pl.when(g < n_groups - 1)
                def _():
                    pltpu.make_async_copy(bufs[b], out_dst(c),
                                          ssems[b]).wait()
                    pltpu.async_copy(gather_src(c + 2), bufs[b], gsems[b])

            return carry

        lax.fori_loop(0, n_groups, group, 0)
        for b in range(2):
            c = (n_groups - 1) * 2 + b
            pltpu.make_async_copy(bufs[b], out_dst(c), ssems[b]).wait()

    return sc_gather


def kernel(slot_ids, piece_ids, orientations, corner_slot_emb,
           corner_piece_emb, corner_orient_emb, edge_slot_emb, edge_piece_emb,
           edge_orient_emb, proj_W, proj_b):
    bsz = slot_ids.shape[0]
    tab = jnp.zeros((48, 128), jnp.float32)
    tab = tab.at[0:8, 0:42].set(corner_slot_emb)
    tab = tab.at[8:16, 42:84].set(corner_piece_emb)
    tab = tab.at[16:19, 84:128].set(corner_orient_emb)
    tab = tab.at[19:31, 0:42].set(edge_slot_emb)
    tab = tab.at[31:43, 42:84].set(edge_piece_emb)
    tab = tab.at[43:45, 84:128].set(edge_orient_emb)
    bias = proj_b.reshape(1, _D)

    lut = pl.pallas_call(
        _lut_body,
        out_shape=jax.ShapeDtypeStruct((_NLUT, _D), jnp.float32),
    )(tab, proj_W, bias)

    tok = jnp.arange(_NT, dtype=jnp.int32)[None, :]
    idx = jnp.where(tok < _NCORNER,
                    slot_ids * 24 + piece_ids * 3 + orientations,
                    192 + slot_ids * 24 + piece_ids * 2 + orientations)

    rows = bsz * _NT
    out = _make_sc_gather(rows)(lut, idx.reshape(rows).astype(jnp.int32))
    return out.reshape(bsz, _NT, _D)
